# Initial kernel scaffold; baseline (speedup 1.0000x reference)
import jax
import jax.numpy as jnp
from jax.experimental import pallas as pl


def kernel(logits, logits_lag):
    raise NotImplementedError("write your pallas kernel here")


# speedup vs baseline: 30.6636x; 30.6636x over previous
import functools

import jax
import jax.numpy as jnp
from jax import lax
from jax.experimental import pallas as pl
from jax.experimental.pallas import tpu as pltpu
from jax.experimental.pallas import tpu_sc as plsc

N = 4096
LAG = 2
N_PAIRS = N * (N - 1) // 2

NW = 32
NBUCK = 8
BH = N // NBUCK

BT = 512
NT = N // BT


def _sc_fill_body(lflat, a_out, s_out,
                  l00, l10, l20, l01, l11, l21,
                  sa0, ss0, sa1, ss1, rsem, wsem):
    wid = lax.axis_index("s") * 2 + lax.axis_index("c")
    lbufs = ((l00, l10, l20), (l01, l11, l21))
    stages = ((sa0, ss0), (sa1, ss1))

    for b in range(NBUCK):
        sz = BH * (b + 1)
        rs = sz + 16
        nv = sz // 16
        clamp_hi = N_PAIRS - rs

        def row_pair(k2, c, b=b, sz=sz, rs=rs, nv=nv, clamp_hi=clamp_hi):
            for half in range(2):
                lb = lbufs[half]
                sta, sts = stages[half]
                k = 2 * k2 + half
                r = BH * b + wid + 32 * k
                off = (r * (r - 1)) // 2
                al = jnp.minimum(off - lax.rem(off, 8), clamp_hi)
                al = pl.multiple_of(al, 8)
                d = off - al

                for p in range(3):
                    pltpu.make_async_copy(
                        lflat.at[pl.ds(p * N_PAIRS + al, rs)],
                        lb[p].at[pl.ds(0, rs)], rsem).start()

                @pl.when(k2 >= 1)
                def _():
                    pltpu.make_async_copy(
                        sta.at[pl.ds(0, sz)],
                        a_out.at[0, pl.ds(0, sz)], wsem).wait()
                    pltpu.make_async_copy(
                        sts.at[pl.ds(0, sz)],
                        a_out.at[0, pl.ds(0, sz)], wsem).wait()

                for p in range(3):
                    pltpu.make_async_copy(
                        lflat.at[pl.ds(p * N_PAIRS + al, rs)],
                        lb[p].at[pl.ds(0, rs)], rsem).wait()

                def vec(m, c2, lb=lb, sta=sta, sts=sts, d=d):
                    e0 = jnp.exp(lb[0][pl.ds(d + m * 16, 16)])
                    e1 = jnp.exp(lb[1][pl.ds(d + m * 16, 16)])
                    e2 = jnp.exp(lb[2][pl.ds(d + m * 16, 16)])
                    inv = 1.0 / (e0 + e1 + e2)
                    sta[pl.ds(m * 16, 16)] = e0 * inv
                    sts[pl.ds(m * 16, 16)] = e1 * inv
                    return c2

                lax.fori_loop(0, nv, vec, 0)

                pltpu.make_async_copy(
                    sta.at[pl.ds(0, sz)],
                    a_out.at[r, pl.ds(0, sz)], wsem).start()
                pltpu.make_async_copy(
                    sts.at[pl.ds(0, sz)],
                    s_out.at[r, pl.ds(0, sz)], wsem).start()
            return c

        lax.fori_loop(0, BH // 64, row_pair, 0)

        for _ in range(4):
            pltpu.make_async_copy(
                sa0.at[pl.ds(0, sz)],
                a_out.at[0, pl.ds(0, sz)], wsem).wait()


def _make_sc_fill():
    mesh = plsc.VectorSubcoreMesh(core_axis_name="c", subcore_axis_name="s")
    return functools.partial(
        pl.kernel, mesh=mesh,
        out_type=(jax.ShapeDtypeStruct((N, N), jnp.float32),
                  jax.ShapeDtypeStruct((N, N), jnp.float32)),
        scratch_types=[
            pltpu.VMEM((N + 32,), jnp.float32),
            pltpu.VMEM((N + 32,), jnp.float32),
            pltpu.VMEM((N + 32,), jnp.float32),
            pltpu.VMEM((N + 32,), jnp.float32),
            pltpu.VMEM((N + 32,), jnp.float32),
            pltpu.VMEM((N + 32,), jnp.float32),
            pltpu.VMEM((N,), jnp.float32),
            pltpu.VMEM((N,), jnp.float32),
            pltpu.VMEM((N,), jnp.float32),
            pltpu.VMEM((N,), jnp.float32),
            pltpu.SemaphoreType.DMA,
            pltpu.SemaphoreType.DMA,
        ],
    )(_sc_fill_body)


_sc_fill_cache = []


def _sc_fill(lflat):
    if not _sc_fill_cache:
        _sc_fill_cache.append(_make_sc_fill())
    return _sc_fill_cache[0](lflat)


def _merge_body(a_ref, st_ref, l0_ref, l1_ref, out_ref):
    p = pl.program_id(0)
    i = pl.program_id(1)
    j = pl.program_id(2)

    @pl.when(p == 0)
    def _():
        st = jnp.swapaxes(st_ref[...], 0, 1)
        a = a_ref[...]
        rows = i * BT + lax.broadcasted_iota(jnp.int32, (BT, BT), 0)
        cols = j * BT + lax.broadcasted_iota(jnp.int32, (BT, BT), 1)
        out_ref[0] = jnp.where(
            cols < rows, a, jnp.where(cols > rows, st, jnp.float32(0.0)))

    @pl.when(p > 0)
    def _():
        out_ref[0] = 1.0 / (1.0 + jnp.exp(l0_ref[0, 0] - l1_ref[0, 0]))


def _zif(p, v):
    return jnp.where(p == 0, v, 0)


_merge = pl.pallas_call(
    _merge_body,
    grid=(3, NT, NT),
    in_specs=[
        pl.BlockSpec((BT, BT), lambda p, i, j: (_zif(p, i), _zif(p, j))),
        pl.BlockSpec((BT, BT), lambda p, i, j: (_zif(p, j), _zif(p, i))),
        pl.BlockSpec(
            (1, 1, BT, BT),
            lambda p, i, j: (0, jnp.maximum(p - 1, 0),
                             jnp.where(p == 0, 0, i), jnp.where(p == 0, 0, j))),
        pl.BlockSpec(
            (1, 1, BT, BT),
            lambda p, i, j: (1, jnp.maximum(p - 1, 0),
                             jnp.where(p == 0, 0, i), jnp.where(p == 0, 0, j))),
    ],
    out_specs=pl.BlockSpec((1, BT, BT), lambda p, i, j: (p, i, j)),
    out_shape=jax.ShapeDtypeStruct((3, N, N), jnp.float32),
)


def kernel(logits, logits_lag):
    a, s = _sc_fill(logits.reshape(-1))
    return _merge(a, s, logits_lag, logits_lag)
